# baseline (speedup 1.0000x reference)
import functools
import jax
import jax.numpy as jnp
from jax import lax
from jax.experimental import pallas as pl
from jax.experimental.pallas import tpu as pltpu
from jax.experimental.pallas import tpu_sc as plsc

_ORDER = tuple(range(17))

_B, _N, _J, _D = 32, 243, 17, 256
_TOTAL = _B * _N * _J * _D
_NC, _NS = 2, 16
_NW = _NC * _NS
_PER_W = _TOTAL // _NW
_NCHUNK = 36
_CHUNK = _PER_W // _NCHUNK
_NBUF = 4
_LOOK = 2


def _make_sc_copy():
    mesh = plsc.VectorSubcoreMesh(
        core_axis_name="c", subcore_axis_name="s",
        num_cores=_NC, num_subcores=_NS)

    @functools.partial(
        pl.kernel,
        mesh=mesh,
        out_type=jax.ShapeDtypeStruct((_TOTAL,), jnp.float32),
        scratch_types=[
            pltpu.VMEM((_CHUNK,), jnp.float32),
            pltpu.VMEM((_CHUNK,), jnp.float32),
            pltpu.VMEM((_CHUNK,), jnp.float32),
            pltpu.VMEM((_CHUNK,), jnp.float32),
            pltpu.SemaphoreType.DMA,
            pltpu.SemaphoreType.DMA,
            pltpu.SemaphoreType.DMA,
            pltpu.SemaphoreType.DMA,
            pltpu.SemaphoreType.DMA,
            pltpu.SemaphoreType.DMA,
            pltpu.SemaphoreType.DMA,
            pltpu.SemaphoreType.DMA,
        ],
    )
    def sc_copy(x_hbm, o_hbm, b0, b1, b2, b3, is0, is1, is2, is3, os0, os1, os2, os3):
        wid = lax.axis_index("s") * _NC + lax.axis_index("c")
        base = wid * _PER_W
        bufs = (b0, b1, b2, b3)
        isems = (is0, is1, is2, is3)
        osems = (os0, os1, os2, os3)

        def in_cp(g):
            k = g % _NBUF
            return pltpu.make_async_copy(
                x_hbm.at[pl.ds(base + g * _CHUNK, _CHUNK)],
                bufs[k], isems[k])

        def out_cp(g):
            k = g % _NBUF
            return pltpu.make_async_copy(
                bufs[k],
                o_hbm.at[pl.ds(base + g * _CHUNK, _CHUNK)], osems[k])

        for j in range(_LOOK):
            in_cp(j).start()
        for g in range(_NCHUNK):
            a = g + _LOOK
            if a < _NCHUNK:
                if a >= _NBUF:
                    out_cp(a - _NBUF).wait()
                in_cp(a).start()
            in_cp(g).wait()
            out_cp(g).start()
        for g in range(_NCHUNK - _NBUF, _NCHUNK):
            out_cp(g).wait()

    return sc_copy


_SC_COPY_CACHE = []


def kernel(x):
    if not _SC_COPY_CACHE:
        _SC_COPY_CACHE.append(_make_sc_copy())
    xt = jax.lax.transpose(x, (1, 2, 0, 3))
    flat = xt.reshape(_TOTAL)
    out = _SC_COPY_CACHE[0](flat)
    out_t = out.reshape(_N, _J, _B, _D)
    return jax.lax.transpose(out_t, (2, 0, 1, 3))


# speedup vs baseline: 3.5221x; 3.5221x over previous
import jax
import jax.numpy as jnp
from jax.experimental import pallas as pl

_ORDER = (0, 1, 2, 3, 4, 5, 6, 7, 8, 9, 10, 11, 12, 13, 14, 15, 16)
_IS_IDENTITY = _ORDER == tuple(range(len(_ORDER)))

_GRID = 81


def _reorder_block(x_ref, o_ref):
    if _IS_IDENTITY:
        o_ref[...] = x_ref[...]
    else:
        for jj, s in enumerate(_ORDER):
            o_ref[:, jj, :, :] = x_ref[:, s, :, :]


def kernel(x):
    b, n, j, d = x.shape
    xt = jax.lax.transpose(x, (1, 2, 0, 3))
    bn = n // _GRID
    out_t = pl.pallas_call(
        _reorder_block,
        grid=(_GRID,),
        in_specs=[pl.BlockSpec((bn, j, b, d), lambda i: (i, 0, 0, 0))],
        out_specs=pl.BlockSpec((bn, j, b, d), lambda i: (i, 0, 0, 0)),
        out_shape=jax.ShapeDtypeStruct((n, j, b, d), x.dtype),
    )(xt)
    return jax.lax.transpose(out_t, (2, 0, 1, 3))
